# Initial kernel scaffold; baseline (speedup 1.0000x reference)
import jax
import jax.numpy as jnp
from jax.experimental import pallas as pl


def kernel(x, edge_index, W_rel1, b_rel1, W_root1, W_rel2, b_rel2, W_root2, W_fc1, b_fc1, W_fc2, b_fc2):
    raise NotImplementedError("write your pallas kernel here")


# speedup vs baseline: 6.3747x; 6.3747x over previous
import functools

import jax
import jax.numpy as jnp
from jax import lax
from jax.experimental import pallas as pl
from jax.experimental.pallas import tpu as pltpu
from jax.experimental.pallas import tpu_sc as plsc

NC = 2
NS = 16
CH = 128


def _sc_aggregate(x, src, dst):
    N, F = x.shape
    E = src.shape[0]
    assert E % CH == 0, E
    total_chunks = E // CH
    NW = NC * NS
    base, rem = divmod(total_chunks, NW)
    BR = 80
    assert N % BR == 0 and BR <= CH
    NBLK = N // BR
    zbase, zrem = divmod(NBLK, NS)

    mesh = plsc.VectorSubcoreMesh(core_axis_name="c", subcore_axis_name="s")

    @functools.partial(
        pl.kernel,
        out_type=jax.ShapeDtypeStruct((NC, N, F), jnp.float32),
        mesh=mesh,
        scratch_types=[
            pltpu.VMEM((CH,), jnp.int32),
            pltpu.VMEM((CH,), jnp.int32),
            pltpu.VMEM((CH, F), jnp.float32),
            pltpu.VMEM_SHARED((N, F), jnp.float32),
            pltpu.SemaphoreType.DMA,
        ],
    )
    def agg_kernel(x_hbm, src_hbm, dst_hbm, out_hbm, src_v, dst_v, rows_v, acc_sh, sem):
        cid = lax.axis_index("c")
        sid = lax.axis_index("s")
        wid = cid * NS + sid

        zero16 = jnp.zeros((16,), jnp.float32)

        def zrow(i, carry):
            for l in range(F // 16):
                rows_v[i, pl.ds(l * 16, 16)] = zero16
            return carry

        lax.fori_loop(0, BR, zrow, 0)
        nzb = zbase + jnp.where(sid < zrem, 1, 0)

        def zblk(t, carry):
            r0 = pl.multiple_of((sid + t * NS) * BR, 8)
            pltpu.sync_copy(rows_v.at[pl.ds(0, BR)], acc_sh.at[pl.ds(r0, BR)])
            return carry

        lax.fori_loop(0, nzb, zblk, 0)
        plsc.subcore_barrier()

        nci = base + jnp.where(wid < rem, 1, 0)

        def body(t, carry):
            off = pl.multiple_of((wid + t * NW) * CH, 8)
            pltpu.sync_copy(src_hbm.at[pl.ds(off, CH)], src_v)
            pltpu.sync_copy(dst_hbm.at[pl.ds(off, CH)], dst_v)
            pltpu.async_copy(x_hbm.at[src_v], rows_v, sem).wait()
            pltpu.sync_copy(rows_v, acc_sh.at[dst_v], add=True)
            return carry

        lax.fori_loop(0, nci, body, 0)
        plsc.subcore_barrier()

        def dblk(t, carry):
            r0 = pl.multiple_of((sid + t * NS) * BR, 8)
            pltpu.sync_copy(acc_sh.at[pl.ds(r0, BR)],
                            out_hbm.at[cid, pl.ds(r0, BR)])
            return carry

        lax.fori_loop(0, nzb, dblk, 0)

    return agg_kernel(x, src, dst)


def _tc_combine(aggp, h, W_rel, W_root, b):
    N, F = h.shape
    H = W_rel.shape[1]
    R = 1000
    G = N // R

    def body(ap_ref, h_ref, wrel_ref, wroot_ref, b_ref, o_ref):
        agg = ap_ref[0] + ap_ref[1]
        acc = jnp.dot(agg, wrel_ref[...], preferred_element_type=jnp.float32)
        acc += jnp.dot(h_ref[...], wroot_ref[...], preferred_element_type=jnp.float32)
        o_ref[...] = jnp.maximum(acc + b_ref[...], 0.0)

    return pl.pallas_call(
        body,
        grid=(G,),
        in_specs=[
            pl.BlockSpec((2, R, F), lambda i: (0, i, 0)),
            pl.BlockSpec((R, F), lambda i: (i, 0)),
            pl.BlockSpec((F, H), lambda i: (0, 0)),
            pl.BlockSpec((F, H), lambda i: (0, 0)),
            pl.BlockSpec((1, H), lambda i: (0, 0)),
        ],
        out_specs=pl.BlockSpec((R, H), lambda i: (i, 0)),
        out_shape=jax.ShapeDtypeStruct((N, H), jnp.float32),
    )(aggp, h, W_rel, W_root, b)


def _tc_final(aggp, h1, W_rel2, W_root2, b_rel2, W_fc1, b_fc1, W_fc2, b_fc2):
    N, H = h1.shape
    C = W_fc2.shape[1]
    R = 1000
    G = N // R

    def body(ap_ref, h1_ref, wrel_ref, wroot_ref, brel_ref,
             wfc1_ref, bfc1_ref, wfc2_ref, bfc2_ref, o_ref):
        agg = ap_ref[0] + ap_ref[1]
        h2 = jnp.dot(agg, wrel_ref[...], preferred_element_type=jnp.float32)
        h2 += jnp.dot(h1_ref[...], wroot_ref[...], preferred_element_type=jnp.float32)
        h2 = jnp.maximum(h2 + brel_ref[...], 0.0)
        h3 = jnp.maximum(
            jnp.dot(h2, wfc1_ref[...], preferred_element_type=jnp.float32)
            + bfc1_ref[...], 0.0)
        o_ref[...] = (jnp.dot(h3, wfc2_ref[...], preferred_element_type=jnp.float32)
                      + bfc2_ref[...])

    return pl.pallas_call(
        body,
        grid=(G,),
        in_specs=[
            pl.BlockSpec((2, R, H), lambda i: (0, i, 0)),
            pl.BlockSpec((R, H), lambda i: (i, 0)),
            pl.BlockSpec((H, H), lambda i: (0, 0)),
            pl.BlockSpec((H, H), lambda i: (0, 0)),
            pl.BlockSpec((1, H), lambda i: (0, 0)),
            pl.BlockSpec((H, H), lambda i: (0, 0)),
            pl.BlockSpec((1, H), lambda i: (0, 0)),
            pl.BlockSpec((H, C), lambda i: (0, 0)),
            pl.BlockSpec((1, C), lambda i: (0, 0)),
        ],
        out_specs=pl.BlockSpec((R, C), lambda i: (i, 0)),
        out_shape=jax.ShapeDtypeStruct((N, C), jnp.float32),
    )(aggp, h1, W_rel2, W_root2, b_rel2, W_fc1, b_fc1, W_fc2, b_fc2)


def kernel(x, edge_index, W_rel1, b_rel1, W_root1, W_rel2, b_rel2, W_root2,
           W_fc1, b_fc1, W_fc2, b_fc2):
    src = edge_index[0]
    dst = edge_index[1]
    agg1 = _sc_aggregate(x, src, dst)
    h1 = _tc_combine(agg1, x, W_rel1, W_root1, b_rel1.reshape(1, -1))
    agg2 = _sc_aggregate(h1, src, dst)
    return _tc_final(agg2, h1, W_rel2, W_root2, b_rel2.reshape(1, -1),
                     W_fc1, b_fc1.reshape(1, -1), W_fc2, b_fc2.reshape(1, -1))
